# baseline (speedup 1.0000x reference)
import jax
import jax.numpy as jnp
from jax import lax
from jax.experimental import pallas as pl
from jax.experimental.pallas import tpu as pltpu
from jax.experimental.pallas import tpu_sc as plsc

_N = 10000
_D = 128
_E = 320000
_NCLS = 40

_NC = 2
_NS = 16
_NW = _NC * _NS
_K = 128
_IB = 16
_CW = 80
_EPAD = _NW * _CW * _K
_KD = 128
_CD = _EPAD // (_NS * _KD)
_NPAD = 10240
_RPT = _NPAD // _NS
_HB = 32


def _pipeline_edges(tab_h, src_h, dst_h, agg_sh, srcb, dstb,
                    rowsA, rowsB, semA, semB, semI, wid, deg_hook):
    nblk = _CW // _IB
    pending = None
    idx_pend = [None, None]
    base0 = wid * _CW
    idx_pend[0] = (
        pltpu.async_copy(src_h.at[pl.ds(base0, _IB), :], srcb[0], semI),
        pltpu.async_copy(dst_h.at[pl.ds(base0, _IB), :], dstb[0], semI),
    )
    for t in range(_CW):
        blk, off = divmod(t, _IB)
        p = blk % 2
        if off == 0:
            idx_pend[p][0].wait()
            idx_pend[p][1].wait()
        sb, db = srcb[p], dstb[p]
        buf, sem = (rowsA, semA) if t % 2 == 0 else (rowsB, semB)
        cp = pltpu.async_copy(tab_h.at[sb.at[off]], buf, sem)
        if deg_hook is not None:
            deg_hook(db.at[off])
        if pending is not None:
            pending[0].wait()
            pltpu.sync_copy(pending[1], agg_sh.at[pending[2]], add=True)
        pending = (cp, buf, db.at[off])
        if off == 1 and blk + 1 < nblk:
            q = (blk + 1) % 2
            base = wid * _CW + (blk + 1) * _IB
            idx_pend[q] = (
                pltpu.async_copy(src_h.at[pl.ds(base, _IB), :], srcb[q],
                                 semI),
                pltpu.async_copy(dst_h.at[pl.ds(base, _IB), :], dstb[q],
                                 semI),
            )
    pending[0].wait()
    pltpu.sync_copy(pending[1], agg_sh.at[pending[2]], add=True)


def _zero_agg(agg_sh, zbuf, r0):
    def _init_zrow(i, carry):
        for j in range(_D // 16):
            zbuf[i, pl.ds(j * 16, 16)] = jnp.zeros((16,), jnp.float32)
        return carry
    lax.fori_loop(0, _HB, _init_zrow, 0)
    for blk in range(_RPT // _HB):
        pltpu.sync_copy(zbuf, agg_sh.at[pl.ds(r0 + blk * _HB, _HB), :])


def _writeout(agg_sh, out_h, c, r0):
    pltpu.sync_copy(agg_sh.at[pl.ds(r0, _RPT), :], out_h.at[c, pl.ds(r0, _RPT), :])


def _mp1_body(tab_h, src_h, dst_h, out_h, degp_h,
              agg_sh, deg_sh, srcb0, srcb1, dstb0, dstb1,
              rowsA, rowsB, ones_v, zbuf, degb, semA, semB, semI):
    c = lax.axis_index("c")
    s = lax.axis_index("s")
    wid = s * _NC + c
    r0 = s * _RPT

    def _init_ones(j, carry):
        ones_v[pl.ds(j * 16, 16)] = jnp.ones((16,), jnp.float32)
        return carry
    lax.fori_loop(0, _K // 16, _init_ones, 0)

    def _zdeg(j, carry):
        degb[pl.ds(j * 16, 16)] = jnp.zeros((16,), jnp.float32)
        return carry
    lax.fori_loop(0, _RPT // 16, _zdeg, 0)

    _zero_agg(agg_sh, zbuf, r0)
    pltpu.sync_copy(degb, deg_sh.at[pl.ds(r0, _RPT)])
    plsc.subcore_barrier()

    def _deg_hook(dref):
        pltpu.sync_copy(ones_v, deg_sh.at[dref], add=True)

    _pipeline_edges(tab_h, src_h, dst_h, agg_sh, (srcb0, srcb1),
                    (dstb0, dstb1), rowsA, rowsB, semA, semB, semI, wid,
                    _deg_hook)

    plsc.subcore_barrier()

    pltpu.sync_copy(deg_sh.at[pl.ds(r0, _RPT)],
                    degp_h.at[c, pl.ds(r0, _RPT)])
    _writeout(agg_sh, out_h, c, r0)


def _mp2_body(tab_h, src_h, dst_h, out_h,
              agg_sh, srcb0, srcb1, dstb0, dstb1, rowsA, rowsB, zbuf,
              semA, semB, semI):
    c = lax.axis_index("c")
    s = lax.axis_index("s")
    wid = s * _NC + c
    r0 = s * _RPT

    _zero_agg(agg_sh, zbuf, r0)
    plsc.subcore_barrier()

    _pipeline_edges(tab_h, src_h, dst_h, agg_sh, (srcb0, srcb1),
                    (dstb0, dstb1), rowsA, rowsB, semA, semB, semI, wid,
                    None)

    plsc.subcore_barrier()

    _writeout(agg_sh, out_h, c, r0)


_sc_mesh = plsc.VectorSubcoreMesh(core_axis_name="c", subcore_axis_name="s")

_idx_scratch = [pltpu.VMEM((_IB, _K), jnp.int32)] * 4
_rows_scratch = [pltpu.VMEM((_K, _D), jnp.float32)] * 2
_sem_scratch = [pltpu.SemaphoreType.DMA] * 3

_mp1_call = pl.kernel(
    _mp1_body,
    out_type=(jax.ShapeDtypeStruct((_NC, _NPAD, _D), jnp.float32),
              jax.ShapeDtypeStruct((_NC, _NPAD), jnp.float32)),
    mesh=_sc_mesh,
    scratch_types=[
        pltpu.VMEM_SHARED((_NPAD, _D), jnp.float32),
        pltpu.VMEM_SHARED((_NPAD,), jnp.float32),
    ] + _idx_scratch + _rows_scratch + [
        pltpu.VMEM((_K,), jnp.float32),
        pltpu.VMEM((_HB, _D), jnp.float32),
        pltpu.VMEM((_RPT,), jnp.float32),
    ] + _sem_scratch,
)

_mp2_call = pl.kernel(
    _mp2_body,
    out_type=jax.ShapeDtypeStruct((_NC, _NPAD, _D), jnp.float32),
    mesh=_sc_mesh,
    scratch_types=[
        pltpu.VMEM_SHARED((_NPAD, _D), jnp.float32),
    ] + _idx_scratch + _rows_scratch + [
        pltpu.VMEM((_HB, _D), jnp.float32),
    ] + _sem_scratch,
)


def _tc1_body(p_ref, dg_ref, w_ref, b_ref, g_ref, be_ref, a_ref, o_ref):
    idg = 1.0 / jnp.maximum(dg_ref[0, :_N, :] + dg_ref[1, :_N, :], 1.0)
    h = (p_ref[0, :_N, :] + p_ref[1, :_N, :]) * idg
    h = jnp.dot(h, w_ref[...], preferred_element_type=jnp.float32) + b_ref[...]
    m = jnp.mean(h, axis=0, keepdims=True)
    v = jnp.mean((h - m) * (h - m), axis=0, keepdims=True)
    h = (h - m) * lax.rsqrt(v + 1e-5) * g_ref[...] + be_ref[...]
    a = a_ref[0, 0]
    o_ref[...] = jnp.where(h > 0, h, a * h)


_tc1_call = pl.pallas_call(
    _tc1_body,
    out_shape=jax.ShapeDtypeStruct((_N, _D), jnp.float32),
)


def _tc2_body(p_ref, dg_ref, w_ref, b_ref, g_ref, be_ref, a_ref, wc_ref,
              bc_ref, o_ref):
    idg = 1.0 / jnp.maximum(dg_ref[0, :_N, :] + dg_ref[1, :_N, :], 1.0)
    h = (p_ref[0, :_N, :] + p_ref[1, :_N, :]) * idg
    h = jnp.dot(h, w_ref[...], preferred_element_type=jnp.float32) + b_ref[...]
    m = jnp.mean(h, axis=0, keepdims=True)
    v = jnp.mean((h - m) * (h - m), axis=0, keepdims=True)
    h = (h - m) * lax.rsqrt(v + 1e-5) * g_ref[...] + be_ref[...]
    a = a_ref[0, 0]
    h = jnp.where(h > 0, h, a * h)
    lg = jnp.dot(h, wc_ref[...], preferred_element_type=jnp.float32)
    lg = lg + bc_ref[...]
    mx = jnp.max(lg, axis=-1, keepdims=True)
    e = jnp.exp(lg - mx)
    o_ref[...] = e / jnp.sum(e, axis=-1, keepdims=True) + 1e-10


_tc2_call = pl.pallas_call(
    _tc2_body,
    out_shape=jax.ShapeDtypeStruct((_N, _NCLS), jnp.float32),
)


def kernel(x, edge_index, W1, b1, g1, be1, a1, W2, b2, g2, be2, a2, Wc, bc):
    src = edge_index[0].astype(jnp.int32)
    dst = edge_index[1].astype(jnp.int32)
    pad = _EPAD - _E
    padidx = lax.iota(jnp.int32, pad)
    srcp = jnp.concatenate([src, padidx % _N])
    dstp = jnp.concatenate([dst, _N + padidx % (_NPAD - _N)])
    src64 = srcp.reshape(_EPAD // _K, _K)
    dst64 = dstp.reshape(_EPAD // _K, _K)

    p1, degp = _mp1_call(x, src64, dst64)
    dgcol = degp.reshape(_NC, _NPAD, 1)
    h1 = _tc1_call(p1, dgcol, W1, b1.reshape(1, _D), g1.reshape(1, _D),
                   be1.reshape(1, _D), a1.reshape(1, 1))
    p2 = _mp2_call(h1, src64, dst64)
    return _tc2_call(p2, dgcol, W2, b2.reshape(1, _D), g2.reshape(1, _D),
                     be2.reshape(1, _D), a2.reshape(1, 1),
                     Wc, bc.reshape(1, _NCLS))


# speedup vs baseline: 1.0185x; 1.0185x over previous
import jax
import jax.numpy as jnp
from jax import lax
from jax.experimental import pallas as pl
from jax.experimental.pallas import tpu as pltpu
from jax.experimental.pallas import tpu_sc as plsc

_N = 10000
_D = 128
_E = 320000
_NCLS = 40

_NC = 2
_NS = 16
_NW = _NC * _NS
_K = 128
_IB = 16
_CW = 80
_EPAD = _NW * _CW * _K
_KD = 128
_CD = _EPAD // (_NS * _KD)
_NPAD = 10240
_RPT = _NPAD // _NS
_HB = 32


def _pipeline_edges(tab_h, src_h, dst_h, agg_sh, srcb, dstb,
                    rowsA, rowsB, semA, semB, semI, wid, deg_hook, prelude):
    nblk = _CW // _IB
    gd = [None] * _CW

    def issue_gather(t):
        blk, off = divmod(t, _IB)
        p = blk % 2
        buf, sem = (rowsA, semA) if t % 2 == 0 else (rowsB, semB)
        gd[t] = (pltpu.async_copy(tab_h.at[srcb[p].at[off]], buf, sem),
                 buf, dstb[p].at[off])

    def scatter(t):
        gd[t][0].wait()
        pltpu.sync_copy(gd[t][1], agg_sh.at[gd[t][2]], add=True)

    def prefetch_idx(blk):
        p = blk % 2
        base = wid * _CW + blk * _IB
        return (pltpu.async_copy(src_h.at[pl.ds(base, _IB), :], srcb[p],
                                 semI),
                pltpu.async_copy(dst_h.at[pl.ds(base, _IB), :], dstb[p],
                                 semI))

    ip = [None] * nblk
    ip[0] = prefetch_idx(0)
    ip[0][0].wait()
    ip[0][1].wait()
    issue_gather(0)
    issue_gather(1)
    prelude()
    for t in range(_CW):
        blk, off = divmod(t, _IB)
        if off == _IB - 2 and blk + 1 < nblk:
            ip[blk + 1][0].wait()
            ip[blk + 1][1].wait()
        scatter(t)
        if t + 2 < _CW:
            issue_gather(t + 2)
        if off == 1 and blk + 1 < nblk:
            ip[blk + 1] = prefetch_idx(blk + 1)
        if deg_hook is not None:
            deg_hook(2 * t)
            deg_hook(2 * t + 1)


def _zero_agg(agg_sh, zbuf, r0):
    def _init_zrow(i, carry):
        for j in range(_D // 16):
            zbuf[i, pl.ds(j * 16, 16)] = jnp.zeros((16,), jnp.float32)
        return carry
    lax.fori_loop(0, _HB, _init_zrow, 0)
    for blk in range(_RPT // _HB):
        pltpu.sync_copy(zbuf, agg_sh.at[pl.ds(r0 + blk * _HB, _HB), :])


def _writeout(agg_sh, out_h, c, r0):
    pltpu.sync_copy(agg_sh.at[pl.ds(r0, _RPT), :], out_h.at[c, pl.ds(r0, _RPT), :])


def _mp1_body(tab_h, src_h, dst_h, dstd_h, out_h, invdeg_h,
              agg_sh, deg_sh, srcb0, srcb1, dstb0, dstb1, degbig,
              rowsA, rowsB, ones_v, zbuf, degb, semA, semB, semI):
    c = lax.axis_index("c")
    s = lax.axis_index("s")
    wid = s * _NC + c
    r0 = s * _RPT

    def _init_ones(j, carry):
        ones_v[pl.ds(j * 16, 16)] = jnp.ones((16,), jnp.float32)
        return carry
    lax.fori_loop(0, _KD // 16, _init_ones, 0)

    def _zdeg(j, carry):
        degb[pl.ds(j * 16, 16)] = jnp.zeros((16,), jnp.float32)
        return carry
    lax.fori_loop(0, _RPT // 16, _zdeg, 0)

    def _prelude():
        _zero_agg(agg_sh, zbuf, r0)
        pltpu.sync_copy(degb, deg_sh.at[pl.ds(r0, _RPT)])
        plsc.subcore_barrier()

    def _deg_hook(d):
        blk, off = divmod(d, _IB)
        if off == 0:
            pltpu.sync_copy(dstd_h.at[pl.ds(s * _CD + blk * _IB, _IB), :],
                            degbig)
        pltpu.sync_copy(ones_v, deg_sh.at[degbig.at[off]], add=True)

    _pipeline_edges(tab_h, src_h, dst_h, agg_sh, (srcb0, srcb1),
                    (dstb0, dstb1), rowsA, rowsB, semA, semB, semI, wid,
                    _deg_hook, _prelude)

    plsc.subcore_barrier()

    pltpu.sync_copy(deg_sh.at[pl.ds(r0, _RPT)], degb)

    def _inv(j, carry):
        dv = degb[pl.ds(j * 16, 16)]
        degb[pl.ds(j * 16, 16)] = 1.0 / jnp.maximum(dv, 1.0)
        return carry
    lax.fori_loop(0, _RPT // 16, _inv, 0)

    @pl.when(c == 0)
    def _():
        pltpu.sync_copy(degb, invdeg_h.at[pl.ds(r0, _RPT)])

    _writeout(agg_sh, out_h, c, r0)


def _mp2_body(tab_h, src_h, dst_h, out_h,
              agg_sh, srcb0, srcb1, dstb0, dstb1, rowsA, rowsB, zbuf,
              semA, semB, semI):
    c = lax.axis_index("c")
    s = lax.axis_index("s")
    wid = s * _NC + c
    r0 = s * _RPT

    def _prelude():
        _zero_agg(agg_sh, zbuf, r0)
        plsc.subcore_barrier()

    _pipeline_edges(tab_h, src_h, dst_h, agg_sh, (srcb0, srcb1),
                    (dstb0, dstb1), rowsA, rowsB, semA, semB, semI, wid,
                    None, _prelude)

    plsc.subcore_barrier()

    _writeout(agg_sh, out_h, c, r0)


_sc_mesh = plsc.VectorSubcoreMesh(core_axis_name="c", subcore_axis_name="s")

_idx_scratch = [pltpu.VMEM((_IB, _K), jnp.int32)] * 4
_rows_scratch = [pltpu.VMEM((_K, _D), jnp.float32)] * 2
_sem_scratch = [pltpu.SemaphoreType.DMA] * 3

_mp1_call = pl.kernel(
    _mp1_body,
    out_type=(jax.ShapeDtypeStruct((_NC, _NPAD, _D), jnp.float32),
              jax.ShapeDtypeStruct((_NPAD,), jnp.float32)),
    mesh=_sc_mesh,
    scratch_types=[
        pltpu.VMEM_SHARED((_NPAD, _D), jnp.float32),
        pltpu.VMEM_SHARED((_NPAD,), jnp.float32),
    ] + _idx_scratch + [
        pltpu.VMEM((_IB, _KD), jnp.int32),
    ] + _rows_scratch + [
        pltpu.VMEM((_KD,), jnp.float32),
        pltpu.VMEM((_HB, _D), jnp.float32),
        pltpu.VMEM((_RPT,), jnp.float32),
    ] + _sem_scratch,
)

_mp2_call = pl.kernel(
    _mp2_body,
    out_type=jax.ShapeDtypeStruct((_NC, _NPAD, _D), jnp.float32),
    mesh=_sc_mesh,
    scratch_types=[
        pltpu.VMEM_SHARED((_NPAD, _D), jnp.float32),
    ] + _idx_scratch + _rows_scratch + [
        pltpu.VMEM((_HB, _D), jnp.float32),
    ] + _sem_scratch,
)


def _tc1_body(p_ref, id_ref, w_ref, b_ref, g_ref, be_ref, a_ref, o_ref):
    h = (p_ref[0, :_N, :] + p_ref[1, :_N, :]) * id_ref[:_N, :]
    h = jnp.dot(h, w_ref[...], preferred_element_type=jnp.float32) + b_ref[...]
    m = jnp.mean(h, axis=0, keepdims=True)
    v = jnp.mean((h - m) * (h - m), axis=0, keepdims=True)
    h = (h - m) * lax.rsqrt(v + 1e-5) * g_ref[...] + be_ref[...]
    a = a_ref[0, 0]
    o_ref[...] = jnp.where(h > 0, h, a * h)


_tc1_call = pl.pallas_call(
    _tc1_body,
    out_shape=jax.ShapeDtypeStruct((_N, _D), jnp.float32),
)


def _tc2_body(p_ref, id_ref, w_ref, b_ref, g_ref, be_ref, a_ref, wc_ref,
              bc_ref, o_ref):
    h = (p_ref[0, :_N, :] + p_ref[1, :_N, :]) * id_ref[:_N, :]
    h = jnp.dot(h, w_ref[...], preferred_element_type=jnp.float32) + b_ref[...]
    m = jnp.mean(h, axis=0, keepdims=True)
    v = jnp.mean((h - m) * (h - m), axis=0, keepdims=True)
    h = (h - m) * lax.rsqrt(v + 1e-5) * g_ref[...] + be_ref[...]
    a = a_ref[0, 0]
    h = jnp.where(h > 0, h, a * h)
    lg = jnp.dot(h, wc_ref[...], preferred_element_type=jnp.float32)
    lg = lg + bc_ref[...]
    mx = jnp.max(lg, axis=-1, keepdims=True)
    e = jnp.exp(lg - mx)
    o_ref[...] = e / jnp.sum(e, axis=-1, keepdims=True) + 1e-10


_tc2_call = pl.pallas_call(
    _tc2_body,
    out_shape=jax.ShapeDtypeStruct((_N, _NCLS), jnp.float32),
)


def kernel(x, edge_index, W1, b1, g1, be1, a1, W2, b2, g2, be2, a2, Wc, bc):
    src = edge_index[0].astype(jnp.int32)
    dst = edge_index[1].astype(jnp.int32)
    pad = _EPAD - _E
    padidx = lax.iota(jnp.int32, pad)
    srcp = jnp.concatenate([src, padidx % _N])
    dstp = jnp.concatenate([dst, _N + padidx % (_NPAD - _N)])
    src64 = srcp.reshape(_EPAD // _K, _K)
    dst64 = dstp.reshape(_EPAD // _K, _K)
    dst128 = dstp.reshape(_EPAD // _KD, _KD)

    p1, invdeg = _mp1_call(x, src64, dst64, dst128)
    idcol = invdeg.reshape(_NPAD, 1)
    h1 = _tc1_call(p1, idcol, W1, b1.reshape(1, _D), g1.reshape(1, _D),
                   be1.reshape(1, _D), a1.reshape(1, 1))
    p2 = _mp2_call(h1, src64, dst64)
    return _tc2_call(p2, idcol, W2, b2.reshape(1, _D), g2.reshape(1, _D),
                     be2.reshape(1, _D), a2.reshape(1, 1),
                     Wc, bc.reshape(1, _NCLS))
